# baseline (speedup 1.0000x reference)
import functools

import jax
import jax.numpy as jnp
from jax import lax
from jax.experimental import pallas as pl
from jax.experimental.pallas import tpu as pltpu
from jax.experimental.pallas import tpu_sc as plsc

B = 1024
N = 100000
D = 64
H = 24
F = 7
HF = H * F
K = 8
NPAD = 100352
NBLK = 2048
NNB = NPAD // NBLK
BBLK = 256
NBB = B // BBLK
CHUNK = 128
CPB = NBLK // CHUNK
NCHUNKS = NPAD // CHUNK
NIDX = B * K
NEG = -1e30

SC_NC = 2
SC_NS = 16
SC_NW = SC_NC * SC_NS


def _norm_keys_body(k_ref, o_ref):
    x = k_ref[...]
    s = jnp.sum(x * x, axis=1, keepdims=True)
    o_ref[...] = (x / jnp.maximum(jnp.sqrt(s), 1e-12)).astype(jnp.bfloat16)


def _sims_body(q_ref, k_ref, sims_ref, cm_ref):
    nb = pl.program_id(0)
    q = q_ref[...]
    qs = jnp.sum(q * q, axis=1, keepdims=True)
    qn = (q / jnp.maximum(jnp.sqrt(qs), 1e-12)).astype(jnp.bfloat16)
    s = lax.dot_general(qn, k_ref[...], (((1,), (1,)), ((), ())),
                        preferred_element_type=jnp.float32)

    @pl.when(nb < NNB - 1)
    def _():
        s3 = s.reshape(BBLK, CPB, CHUNK)
        sims_ref[...] = s3
        cm_ref[0, 0] = jnp.max(s3, axis=-1)

    @pl.when(nb == NNB - 1)
    def _():
        col = nb * NBLK + lax.broadcasted_iota(jnp.int32, (BBLK, NBLK), 1)
        s3 = jnp.where(col < N, s, NEG).reshape(BBLK, CPB, CHUNK)
        sims_ref[...] = s3
        cm_ref[0, 0] = jnp.max(s3, axis=-1)


def _chunktop_body(cm_ref, win_ref, flat_ref):
    cm = cm_ref[...]
    iota = lax.broadcasted_iota(jnp.int32, (B, NCHUNKS), 1)
    poss = []
    for _ in range(K):
        m = jnp.max(cm, axis=1, keepdims=True)
        eq = cm == m
        pos = jnp.min(jnp.where(eq, iota, NCHUNKS), axis=1, keepdims=True)
        poss.append(pos)
        cm = jnp.where(iota == pos, NEG, cm)
    win = jnp.concatenate(poss, axis=1)
    win_ref[...] = win
    rows = lax.broadcasted_iota(jnp.int32, (B, K), 0)
    flat_ref[...] = win + NCHUNKS * rows


def _final_body(cand_ref, win_ref, ts_ref, ti_ref):
    c = cand_ref[...]
    win = win_ref[...]
    width = K * CHUNK
    iota = lax.broadcasted_iota(jnp.int32, (B, width), 1)
    vals, gids = [], []
    for _ in range(K):
        m = jnp.max(c, axis=1, keepdims=True)
        eq = c == m
        pos = jnp.min(jnp.where(eq, iota, width), axis=1, keepdims=True)
        sel = pos // CHUNK
        off = pos % CHUNK
        base = jnp.zeros_like(pos)
        for j in range(K):
            base = base + jnp.where(sel == j, win[:, j:j + 1], 0)
        vals.append(m)
        gids.append(base * CHUNK + off)
        c = jnp.where(iota == pos, NEG, c)
    ts_ref[...] = jnp.concatenate(vals, axis=1)
    ti_ref[...] = jnp.concatenate(gids, axis=1)


def _sc_row_gather(table, idx, rows, cols, tc_tiling=True):
    bpw = rows // SC_NW
    mesh = plsc.VectorSubcoreMesh(core_axis_name="c", subcore_axis_name="s")

    @functools.partial(
        pl.kernel,
        out_type=jax.ShapeDtypeStruct((rows, cols), jnp.float32),
        mesh=mesh,
        scratch_types=[
            pltpu.VMEM((bpw,), jnp.int32),
            pltpu.VMEM((bpw, cols), jnp.float32),
            pltpu.SemaphoreType.DMA,
        ],
        compiler_params=pltpu.CompilerParams(use_tc_tiling_on_sc=tc_tiling),
    )
    def k(table_hbm, idx_hbm, out_hbm, idx_v, rows_v, sem):
        wid = lax.axis_index("s") * SC_NC + lax.axis_index("c")
        base = wid * bpw
        pltpu.sync_copy(idx_hbm.at[pl.ds(base, bpw)], idx_v)
        pltpu.async_copy(table_hbm.at[idx_v], rows_v, sem).wait()
        pltpu.sync_copy(rows_v, out_hbm.at[pl.ds(base, bpw)])

    return k(table, idx)


def _sc_col_gather(table_t, idx):
    mesh = plsc.VectorSubcoreMesh(core_axis_name="c", subcore_axis_name="s")
    n_t = (HF + SC_NW - 1) // SC_NW

    @functools.partial(
        pl.kernel,
        out_type=jax.ShapeDtypeStruct((HF, NIDX), jnp.float32),
        mesh=mesh,
        scratch_types=[
            pltpu.VMEM((NIDX,), jnp.int32),
            pltpu.VMEM((NIDX,), jnp.float32),
            pltpu.SemaphoreType.DMA,
        ],
        compiler_params=pltpu.CompilerParams(use_tc_tiling_on_sc=False),
    )
    def k(t_hbm, idx_hbm, out_hbm, idx_v, orow_v, sem):
        wid = lax.axis_index("s") * SC_NC + lax.axis_index("c")
        pltpu.sync_copy(idx_hbm, idx_v)

        def outer(t, carry):
            r = t * SC_NW + wid

            @pl.when(r < HF)
            def _():
                pltpu.async_copy(t_hbm.at[r].at[idx_v], orow_v, sem).wait()
                pltpu.sync_copy(orow_v, out_hbm.at[r])

            return carry

        lax.fori_loop(0, n_t, outer, 0)

    return k(table_t, idx)


def _tc_pipeline(query, kpad, interpret=False):
    kn = pl.pallas_call(
        _norm_keys_body,
        grid=(NNB,),
        in_specs=[pl.BlockSpec((NBLK, D), lambda i: (i, 0))],
        out_specs=pl.BlockSpec((NBLK, D), lambda i: (i, 0)),
        out_shape=jax.ShapeDtypeStruct((NPAD, D), jnp.bfloat16),
        interpret=interpret,
    )(kpad)

    sims, cm4 = pl.pallas_call(
        _sims_body,
        grid=(NNB, NBB),
        in_specs=[
            pl.BlockSpec((BBLK, D), lambda nb, bb: (bb, 0)),
            pl.BlockSpec((NBLK, D), lambda nb, bb: (nb, 0)),
        ],
        out_specs=[
            pl.BlockSpec((BBLK, CPB, CHUNK), lambda nb, bb: (bb, nb, 0)),
            pl.BlockSpec((1, 1, BBLK, CPB), lambda nb, bb: (nb, bb, 0, 0)),
        ],
        out_shape=[
            jax.ShapeDtypeStruct((B, NCHUNKS, CHUNK), jnp.float32),
            jax.ShapeDtypeStruct((NNB, NBB, BBLK, CPB), jnp.float32),
        ],
        interpret=interpret,
    )(query, kn)

    cm = cm4.transpose(1, 2, 0, 3).reshape(B, NCHUNKS)

    win, flat = pl.pallas_call(
        _chunktop_body,
        in_specs=[pl.BlockSpec((B, NCHUNKS), lambda: (0, 0))],
        out_specs=[
            pl.BlockSpec((B, K), lambda: (0, 0)),
            pl.BlockSpec((B, K), lambda: (0, 0)),
        ],
        out_shape=[
            jax.ShapeDtypeStruct((B, K), jnp.int32),
            jax.ShapeDtypeStruct((B, K), jnp.int32),
        ],
        interpret=interpret,
    )(cm)
    return sims, win, flat


def _tc_final(cand, win, interpret=False):
    return pl.pallas_call(
        _final_body,
        in_specs=[
            pl.BlockSpec((B, K * CHUNK), lambda: (0, 0)),
            pl.BlockSpec((B, K), lambda: (0, 0)),
        ],
        out_specs=[
            pl.BlockSpec((B, K), lambda: (0, 0)),
            pl.BlockSpec((B, K), lambda: (0, 0)),
        ],
        out_shape=[
            jax.ShapeDtypeStruct((B, K), jnp.float32),
            jax.ShapeDtypeStruct((B, K), jnp.int32),
        ],
        interpret=interpret,
    )(cand, win)


def kernel(query, mem_keys, mem_values, top_k):
    kpad = jnp.pad(mem_keys, ((0, NPAD - N), (0, 0)))
    sims, win, flat = _tc_pipeline(query, kpad)

    cand = _sc_row_gather(sims.reshape(B * NCHUNKS, CHUNK),
                          flat.reshape(NIDX), NIDX, CHUNK, tc_tiling=True)
    top_sims, top_idx = _tc_final(cand.reshape(B, K * CHUNK), win)

    vals_t = mem_values.transpose(1, 2, 0).reshape(HF, N)
    out_t = _sc_col_gather(vals_t, top_idx.reshape(NIDX))
    retrieved = out_t.reshape(H, F, B, K).transpose(2, 3, 0, 1)
    valid_mask = top_sims >= 0.0
    return retrieved, top_sims, valid_mask


# speedup vs baseline: 1.6900x; 1.6900x over previous
import functools

import jax
import jax.numpy as jnp
from jax import lax
from jax.experimental import pallas as pl
from jax.experimental.pallas import tpu as pltpu
from jax.experimental.pallas import tpu_sc as plsc

B = 1024
N = 100000
D = 64
H = 24
F = 7
HF = H * F
K = 8
NPAD = 100352
NBLK = 2048
NNB = NPAD // NBLK
BBLK = 256
NBB = B // BBLK
CHUNK = 128
CPB = NBLK // CHUNK
NCHUNKS = NPAD // CHUNK
NIDX = B * K
NEG = -1e30

SC_NC = 2
SC_NS = 16
SC_NW = SC_NC * SC_NS


def _norm_keys_body(k_ref, o_ref):
    x = k_ref[...]
    s = jnp.sum(x * x, axis=1, keepdims=True)
    o_ref[...] = (x / jnp.maximum(jnp.sqrt(s), 1e-12)).astype(jnp.bfloat16)


def _sims_body(q_ref, k_ref, sims_ref, cm_ref):
    nb = pl.program_id(0)
    q = q_ref[...]
    qs = jnp.sum(q * q, axis=1, keepdims=True)
    qn = (q / jnp.maximum(jnp.sqrt(qs), 1e-12)).astype(jnp.bfloat16)
    s = lax.dot_general(qn, k_ref[...], (((1,), (1,)), ((), ())),
                        preferred_element_type=jnp.float32)

    @pl.when(nb < NNB - 1)
    def _():
        s3 = s.reshape(BBLK, CPB, CHUNK)
        sims_ref[...] = s3
        cm_ref[0, 0] = jnp.max(s3, axis=-1)

    @pl.when(nb == NNB - 1)
    def _():
        col = nb * NBLK + lax.broadcasted_iota(jnp.int32, (BBLK, NBLK), 1)
        s3 = jnp.where(col < N, s, NEG).reshape(BBLK, CPB, CHUNK)
        sims_ref[...] = s3
        cm_ref[0, 0] = jnp.max(s3, axis=-1)


def _chunktop_body(cm_ref, win_ref, flat_ref):
    cm = cm_ref[...]
    iota = lax.broadcasted_iota(jnp.int32, (B, NCHUNKS), 1)
    poss = []
    for _ in range(K):
        m = jnp.max(cm, axis=1, keepdims=True)
        eq = cm == m
        pos = jnp.min(jnp.where(eq, iota, NCHUNKS), axis=1, keepdims=True)
        poss.append(pos)
        cm = jnp.where(iota == pos, NEG, cm)
    win = jnp.concatenate(poss, axis=1)
    win_ref[...] = win
    rows = lax.broadcasted_iota(jnp.int32, (B, K), 0)
    flat_ref[...] = win + NCHUNKS * rows


def _final_body(cand_ref, win_ref, ts_ref, ti_ref):
    c = cand_ref[...]
    win = win_ref[...]
    width = K * CHUNK
    iota = lax.broadcasted_iota(jnp.int32, (B, width), 1)
    vals, gids = [], []
    for _ in range(K):
        m = jnp.max(c, axis=1, keepdims=True)
        eq = c == m
        pos = jnp.min(jnp.where(eq, iota, width), axis=1, keepdims=True)
        sel = pos // CHUNK
        off = pos % CHUNK
        base = jnp.zeros_like(pos)
        for j in range(K):
            base = base + jnp.where(sel == j, win[:, j:j + 1], 0)
        vals.append(m)
        gids.append(base * CHUNK + off)
        c = jnp.where(iota == pos, NEG, c)
    ts_ref[...] = jnp.concatenate(vals, axis=1)
    ti_ref[...] = jnp.concatenate(gids, axis=1)


def _sc_row_gather(table, idx, rows, cols, tc_tiling=True):
    bpw = rows // SC_NW
    mesh = plsc.VectorSubcoreMesh(core_axis_name="c", subcore_axis_name="s")

    @functools.partial(
        pl.kernel,
        out_type=jax.ShapeDtypeStruct((rows, cols), jnp.float32),
        mesh=mesh,
        scratch_types=[
            pltpu.VMEM((bpw,), jnp.int32),
            pltpu.VMEM((bpw, cols), jnp.float32),
            pltpu.SemaphoreType.DMA,
        ],
        compiler_params=pltpu.CompilerParams(use_tc_tiling_on_sc=tc_tiling),
    )
    def k(table_hbm, idx_hbm, out_hbm, idx_v, rows_v, sem):
        wid = lax.axis_index("s") * SC_NC + lax.axis_index("c")
        base = wid * bpw
        pltpu.sync_copy(idx_hbm.at[pl.ds(base, bpw)], idx_v)
        pltpu.async_copy(table_hbm.at[idx_v], rows_v, sem).wait()
        pltpu.sync_copy(rows_v, out_hbm.at[pl.ds(base, bpw)])

    return k(table, idx)


def _sc_col_gather(table_t, idx):
    mesh = plsc.VectorSubcoreMesh(core_axis_name="c", subcore_axis_name="s")
    n_t = (HF + SC_NW - 1) // SC_NW

    @functools.partial(
        pl.kernel,
        out_type=jax.ShapeDtypeStruct((HF, NIDX), jnp.float32),
        mesh=mesh,
        scratch_types=[
            pltpu.VMEM((NIDX,), jnp.int32),
            pltpu.VMEM((NIDX,), jnp.float32),
            pltpu.SemaphoreType.DMA,
        ],
        compiler_params=pltpu.CompilerParams(use_tc_tiling_on_sc=False),
    )
    def k(t_hbm, idx_hbm, out_hbm, idx_v, orow_v, sem):
        wid = lax.axis_index("s") * SC_NC + lax.axis_index("c")
        pltpu.sync_copy(idx_hbm, idx_v)

        def outer(t, carry):
            r = t * SC_NW + wid

            @pl.when(r < HF)
            def _():
                pltpu.async_copy(t_hbm.at[r].at[idx_v], orow_v, sem).wait()
                pltpu.sync_copy(orow_v, out_hbm.at[r])

            return carry

        lax.fori_loop(0, n_t, outer, 0)

    return k(table_t, idx)


def _tc_pipeline(query, kpad, interpret=False):
    kn = pl.pallas_call(
        _norm_keys_body,
        grid=(NNB,),
        in_specs=[pl.BlockSpec((NBLK, D), lambda i: (i, 0))],
        out_specs=pl.BlockSpec((NBLK, D), lambda i: (i, 0)),
        out_shape=jax.ShapeDtypeStruct((NPAD, D), jnp.bfloat16),
        interpret=interpret,
    )(kpad)

    sims, cm4 = pl.pallas_call(
        _sims_body,
        grid=(NNB, NBB),
        in_specs=[
            pl.BlockSpec((BBLK, D), lambda nb, bb: (bb, 0)),
            pl.BlockSpec((NBLK, D), lambda nb, bb: (nb, 0)),
        ],
        out_specs=[
            pl.BlockSpec((BBLK, CPB, CHUNK), lambda nb, bb: (bb, nb, 0)),
            pl.BlockSpec((1, 1, BBLK, CPB), lambda nb, bb: (nb, bb, 0, 0)),
        ],
        out_shape=[
            jax.ShapeDtypeStruct((B, NCHUNKS, CHUNK), jnp.float32),
            jax.ShapeDtypeStruct((NNB, NBB, BBLK, CPB), jnp.float32),
        ],
        interpret=interpret,
    )(query, kn)

    cm = cm4.transpose(1, 2, 0, 3).reshape(B, NCHUNKS)

    win, flat = pl.pallas_call(
        _chunktop_body,
        in_specs=[pl.BlockSpec((B, NCHUNKS), lambda: (0, 0))],
        out_specs=[
            pl.BlockSpec((B, K), lambda: (0, 0)),
            pl.BlockSpec((B, K), lambda: (0, 0)),
        ],
        out_shape=[
            jax.ShapeDtypeStruct((B, K), jnp.int32),
            jax.ShapeDtypeStruct((B, K), jnp.int32),
        ],
        interpret=interpret,
    )(cm)
    return sims, win, flat


def _tc_final(cand, win, interpret=False):
    return pl.pallas_call(
        _final_body,
        in_specs=[
            pl.BlockSpec((B, K * CHUNK), lambda: (0, 0)),
            pl.BlockSpec((B, K), lambda: (0, 0)),
        ],
        out_specs=[
            pl.BlockSpec((B, K), lambda: (0, 0)),
            pl.BlockSpec((B, K), lambda: (0, 0)),
        ],
        out_shape=[
            jax.ShapeDtypeStruct((B, K), jnp.float32),
            jax.ShapeDtypeStruct((B, K), jnp.int32),
        ],
        interpret=interpret,
    )(cand, win)


def kernel(query, mem_keys, mem_values, top_k):
    kpad = jnp.pad(mem_keys, ((0, NPAD - N), (0, 0)))
    sims, win, flat = _tc_pipeline(query, kpad)

    cand = _sc_row_gather(sims.reshape(B * NCHUNKS, CHUNK),
                          flat.reshape(NIDX), NIDX, CHUNK, tc_tiling=True)
    top_sims, top_idx = _tc_final(cand.reshape(B, K * CHUNK), win)

    vals = _sc_row_gather(mem_values.reshape(N, HF),
                          top_idx.reshape(NIDX), NIDX, HF, tc_tiling=False)
    retrieved = vals.reshape(B, K, H, F)
    valid_mask = top_sims >= 0.0
    return retrieved, top_sims, valid_mask


# speedup vs baseline: 1.6908x; 1.0005x over previous
import functools

import jax
import jax.numpy as jnp
from jax import lax
from jax.experimental import pallas as pl
from jax.experimental.pallas import tpu as pltpu
from jax.experimental.pallas import tpu_sc as plsc

B = 1024
N = 100000
D = 64
H = 24
F = 7
HF = H * F
K = 8
NPAD = 100352
NBLK = 2048
NNB = NPAD // NBLK
BBLK = 256
NBB = B // BBLK
CHUNK = 128
CPB = NBLK // CHUNK
NCHUNKS = NPAD // CHUNK
NIDX = B * K
NEG = -1e30

SC_NC = 2
SC_NS = 16
SC_NW = SC_NC * SC_NS


def _norm_keys_body(k_ref, o_ref):
    x = k_ref[...]
    s = jnp.sum(x * x, axis=1, keepdims=True)
    o_ref[...] = (x / jnp.maximum(jnp.sqrt(s), 1e-12)).astype(jnp.bfloat16)


def _sims_body(q_ref, k_ref, sims_ref, cm_ref):
    nb = pl.program_id(0)
    q = q_ref[...]
    qs = jnp.sum(q * q, axis=1, keepdims=True)
    qn = (q / jnp.maximum(jnp.sqrt(qs), 1e-12)).astype(jnp.bfloat16)
    s = lax.dot_general(qn, k_ref[...], (((1,), (1,)), ((), ())),
                        preferred_element_type=jnp.float32)

    @pl.when(nb < NNB - 1)
    def _():
        s3 = s.reshape(BBLK, CPB, CHUNK)
        sims_ref[...] = s3
        cm_ref[0, 0] = jnp.max(s3, axis=-1)

    @pl.when(nb == NNB - 1)
    def _():
        col = nb * NBLK + lax.broadcasted_iota(jnp.int32, (BBLK, NBLK), 1)
        s3 = jnp.where(col < N, s, NEG).reshape(BBLK, CPB, CHUNK)
        sims_ref[...] = s3
        cm_ref[0, 0] = jnp.max(s3, axis=-1)


def _chunktop_body(cm_ref, win_ref, flat_ref):
    cm = cm_ref[...]
    iota = lax.broadcasted_iota(jnp.int32, (B, NCHUNKS), 1)
    poss = []
    for _ in range(K):
        m = jnp.max(cm, axis=1, keepdims=True)
        eq = cm == m
        pos = jnp.min(jnp.where(eq, iota, NCHUNKS), axis=1, keepdims=True)
        poss.append(pos)
        cm = jnp.where(iota == pos, NEG, cm)
    win = jnp.concatenate(poss, axis=1)
    win_ref[...] = win
    rows = lax.broadcasted_iota(jnp.int32, (B, K), 0)
    flat_ref[...] = win + NCHUNKS * rows


def _final_body(cand_ref, win_ref, ts_ref, ti_ref):
    c = cand_ref[...]
    win = win_ref[...]
    width = K * CHUNK
    iota = lax.broadcasted_iota(jnp.int32, (B, width), 1)
    vals, gids = [], []
    for _ in range(K):
        m = jnp.max(c, axis=1, keepdims=True)
        eq = c == m
        pos = jnp.min(jnp.where(eq, iota, width), axis=1, keepdims=True)
        sel = pos // CHUNK
        off = pos % CHUNK
        base = jnp.zeros_like(pos)
        for j in range(K):
            base = base + jnp.where(sel == j, win[:, j:j + 1], 0)
        vals.append(m)
        gids.append(base * CHUNK + off)
        c = jnp.where(iota == pos, NEG, c)
    ts_ref[...] = jnp.concatenate(vals, axis=1)
    ti_ref[...] = jnp.concatenate(gids, axis=1)


def _sc_row_gather(table, idx, rows, cols, tc_tiling=True):
    bpw = rows // SC_NW
    mesh = plsc.VectorSubcoreMesh(core_axis_name="c", subcore_axis_name="s")

    @functools.partial(
        pl.kernel,
        out_type=jax.ShapeDtypeStruct((rows, cols), jnp.float32),
        mesh=mesh,
        scratch_types=[
            pltpu.VMEM((bpw,), jnp.int32),
            pltpu.VMEM((bpw, cols), jnp.float32),
            pltpu.SemaphoreType.DMA,
        ],
        compiler_params=pltpu.CompilerParams(use_tc_tiling_on_sc=tc_tiling),
    )
    def k(table_hbm, idx_hbm, out_hbm, idx_v, rows_v, sem):
        wid = lax.axis_index("s") * SC_NC + lax.axis_index("c")
        base = wid * bpw
        pltpu.sync_copy(idx_hbm.at[pl.ds(base, bpw)], idx_v)
        pltpu.async_copy(table_hbm.at[idx_v], rows_v, sem).wait()
        pltpu.sync_copy(rows_v, out_hbm.at[pl.ds(base, bpw)])

    return k(table, idx)


def _tc_pipeline(query, kpad, interpret=False):
    kn = pl.pallas_call(
        _norm_keys_body,
        grid=(NNB,),
        in_specs=[pl.BlockSpec((NBLK, D), lambda i: (i, 0))],
        out_specs=pl.BlockSpec((NBLK, D), lambda i: (i, 0)),
        out_shape=jax.ShapeDtypeStruct((NPAD, D), jnp.bfloat16),
        interpret=interpret,
    )(kpad)

    sims, cm4 = pl.pallas_call(
        _sims_body,
        grid=(NNB, NBB),
        in_specs=[
            pl.BlockSpec((BBLK, D), lambda nb, bb: (bb, 0)),
            pl.BlockSpec((NBLK, D), lambda nb, bb: (nb, 0)),
        ],
        out_specs=[
            pl.BlockSpec((BBLK, CPB, CHUNK), lambda nb, bb: (bb, nb, 0)),
            pl.BlockSpec((1, 1, BBLK, CPB), lambda nb, bb: (nb, bb, 0, 0)),
        ],
        out_shape=[
            jax.ShapeDtypeStruct((B, NCHUNKS, CHUNK), jnp.float32),
            jax.ShapeDtypeStruct((NNB, NBB, BBLK, CPB), jnp.float32),
        ],
        interpret=interpret,
    )(query, kn)

    cm = cm4.transpose(1, 2, 0, 3).reshape(B, NCHUNKS)

    win, flat = pl.pallas_call(
        _chunktop_body,
        in_specs=[pl.BlockSpec((B, NCHUNKS), lambda: (0, 0))],
        out_specs=[
            pl.BlockSpec((B, K), lambda: (0, 0)),
            pl.BlockSpec((B, K), lambda: (0, 0)),
        ],
        out_shape=[
            jax.ShapeDtypeStruct((B, K), jnp.int32),
            jax.ShapeDtypeStruct((B, K), jnp.int32),
        ],
        interpret=interpret,
    )(cm)
    return sims, win, flat


def _tc_final(cand, win, interpret=False):
    return pl.pallas_call(
        _final_body,
        in_specs=[
            pl.BlockSpec((B, K * CHUNK), lambda: (0, 0)),
            pl.BlockSpec((B, K), lambda: (0, 0)),
        ],
        out_specs=[
            pl.BlockSpec((B, K), lambda: (0, 0)),
            pl.BlockSpec((B, K), lambda: (0, 0)),
        ],
        out_shape=[
            jax.ShapeDtypeStruct((B, K), jnp.float32),
            jax.ShapeDtypeStruct((B, K), jnp.int32),
        ],
        interpret=interpret,
    )(cand, win)


def kernel(query, mem_keys, mem_values, top_k):
    kpad = jnp.pad(mem_keys, ((0, NPAD - N), (0, 0)))
    sims, win, flat = _tc_pipeline(query, kpad)

    cand = _sc_row_gather(sims.reshape(B * NCHUNKS, CHUNK),
                          flat.reshape(NIDX), NIDX, CHUNK, tc_tiling=True)
    top_sims, top_idx = _tc_final(cand.reshape(B, K * CHUNK), win)

    vals = _sc_row_gather(mem_values.reshape(N, HF),
                          top_idx.reshape(NIDX), NIDX, HF, tc_tiling=False)
    retrieved = vals.reshape(B, K, H, F)
    valid_mask = top_sims >= 0.0
    return retrieved, top_sims, valid_mask
